# baseline (speedup 1.0000x reference)
import functools

import jax
import jax.numpy as jnp
from jax import lax
from jax.experimental import pallas as pl
from jax.experimental.pallas import tpu as pltpu
from jax.experimental.pallas import tpu_sc as plsc

_B, _D, _C, _E = 4096, 1000, 40, 64
_K = 8
_CP = 48
_NC, _NS = 2, 16
_NW = _NC * _NS
_QPW = _B // _NW
_BB = 256
_NEG = -1e30


def _score_topk_body(dfea_ref, dp_ref, v1_ref, u1_ref, dsim_ref, idx_ref):
    f32 = jnp.float32
    m_mat = lax.dot_general(u1_ref[...], v1_ref[...], (((0,), (0,)), ((), ())),
                            preferred_element_type=f32)
    g = jnp.dot(dfea_ref[...], m_mat, preferred_element_type=f32)
    p = lax.dot_general(g, dp_ref[...], (((1,), (1,)), ((), ())),
                        preferred_element_type=f32)
    p_max = jnp.max(p, axis=1, keepdims=True)
    p_exp = jnp.exp(p - p_max)
    p_sum = jnp.sum(p_exp, axis=1, keepdims=True)
    pd = p_exp / p_sum
    iota = lax.broadcasted_iota(jnp.int32, pd.shape, 1)
    vals, idxs = [], []
    for _ in range(_K):
        mv = jnp.max(pd, axis=1, keepdims=True)
        cand = jnp.where(pd == mv, iota, _D)
        am = jnp.min(cand, axis=1, keepdims=True)
        vals.append(mv)
        idxs.append(am)
        pd = jnp.where(iota == am, -1.0, pd)
    dsim_ref[...] = jnp.concatenate(vals, axis=1)
    idx_ref[...] = jnp.concatenate(idxs, axis=1)


_score_topk = pl.pallas_call(
    _score_topk_body,
    grid=(_B // _BB,),
    in_specs=[
        pl.BlockSpec((_BB, _E), lambda i: (i, 0)),
        pl.BlockSpec((_D, _E), lambda i: (0, 0)),
        pl.BlockSpec((32, _E), lambda i: (0, 0)),
        pl.BlockSpec((32, _E), lambda i: (0, 0)),
    ],
    out_specs=[
        pl.BlockSpec((_BB, _K), lambda i: (i, 0)),
        pl.BlockSpec((_BB, _K), lambda i: (i, 0)),
    ],
    out_shape=[
        jax.ShapeDtypeStruct((_B, _K), jnp.float32),
        jax.ShapeDtypeStruct((_B, _K), jnp.int32),
    ],
)


def _classify_body(cp_hbm, idx_hbm, ds_hbm, cf_hbm, out_hbm,
                   idxv, dsv, cfv, rowbuf, outv, sem):
    f32 = jnp.float32
    wid = lax.axis_index("s") * _NC + lax.axis_index("c")
    base = wid * _QPW
    pltpu.sync_copy(idx_hbm.at[pl.ds(base, _QPW)], idxv)
    pltpu.sync_copy(ds_hbm.at[pl.ds(base, _QPW)], dsv)
    pltpu.sync_copy(cf_hbm.at[pl.ds(base, _QPW)], cfv)

    lane = lax.iota(jnp.int32, 16)
    cbase = [lane * _E, (lane + 16) * _E, jnp.minimum(lane + 32, _C - 1) * _E]
    padmask = lane >= (_C - 32)
    zero16 = jnp.zeros((16,), f32)

    def q_body(q, carry):
        cp = pltpu.make_async_copy(cp_hbm.at[idxv.at[q]], rowbuf, sem)
        cp.start()
        cp.wait()

        def n_body(n, oacc):
            nvec = jnp.full((16,), n, jnp.int32)
            acc = [zero16, zero16, zero16]
            for ch in range(_E // 16):
                cf_chunk = cfv[q, pl.ds(ch * 16, 16)]
                for k in range(16):
                    e = ch * 16 + k
                    ce = cf_chunk.at[jnp.full((16,), k, jnp.int32)].get(
                        mode="promise_in_bounds")
                    for j in range(3):
                        g = plsc.load_gather(rowbuf, [nvec, cbase[j] + e])
                        acc[j] = acc[j] + g * ce
            a2 = jnp.where(padmask, _NEG, acc[2])
            m = jnp.maximum(jnp.maximum(jnp.max(acc[0]), jnp.max(acc[1])),
                            jnp.max(a2))
            e0 = jnp.exp(acc[0] - m)
            e1 = jnp.exp(acc[1] - m)
            e2 = jnp.exp(a2 - m)
            s = jnp.sum(e0) + jnp.sum(e1) + jnp.sum(e2)
            ds_row = dsv[q, pl.ds(0, 16)]
            dsb = ds_row.at[jnp.full((16,), n, jnp.int32)].get(
                mode="promise_in_bounds")
            w = dsb / jnp.full((16,), s, f32)
            return (oacc[0] + w * e0, oacc[1] + w * e1, oacc[2] + w * e2)

        oa = lax.fori_loop(0, _K, n_body, (zero16, zero16, zero16))
        o2 = jnp.where(padmask, _NEG, oa[2])
        m = jnp.maximum(jnp.maximum(jnp.max(oa[0]), jnp.max(oa[1])), jnp.max(o2))
        f0 = jnp.exp(oa[0] - m)
        f1 = jnp.exp(oa[1] - m)
        f2 = jnp.exp(o2 - m)
        s = jnp.sum(f0) + jnp.sum(f1) + jnp.sum(f2)
        outv[q, pl.ds(0, 16)] = f0 / s
        outv[q, pl.ds(16, 16)] = f1 / s
        outv[q, pl.ds(32, 16)] = f2 / s
        return carry

    lax.fori_loop(0, _QPW, q_body, 0)
    pltpu.sync_copy(outv, out_hbm.at[pl.ds(base, _QPW)])


@functools.cache
def _make_classify():
    return pl.kernel(
        _classify_body,
        mesh=plsc.VectorSubcoreMesh(core_axis_name="c", subcore_axis_name="s"),
        out_type=jax.ShapeDtypeStruct((_B, _CP), jnp.float32),
        compiler_params=pltpu.CompilerParams(
            use_tc_tiling_on_sc=False, needs_layout_passes=False),
        scratch_types=[
            pltpu.VMEM((_QPW, _K), jnp.int32),
            pltpu.VMEM((_QPW, 16), jnp.float32),
            pltpu.VMEM((_QPW, _E), jnp.float32),
            pltpu.VMEM((_K, _C * _E), jnp.float32),
            pltpu.VMEM((_QPW, _CP), jnp.float32),
            pltpu.SemaphoreType.DMA,
        ],
    )


def kernel(c_fea, d_fea, domain_p, class_p, V1, U1, num):
    del num
    dsim, idx = _score_topk(d_fea, domain_p, V1, U1)
    dsim16 = jnp.concatenate(
        [dsim, jnp.zeros((_B, 16 - _K), jnp.float32)], axis=1)
    cp2d = class_p.reshape(_D, _C * _E)
    out48 = _make_classify()(cp2d, idx, dsim16, c_fea)
    return out48[:, :_C]


# speedup vs baseline: 1.1370x; 1.1370x over previous
import functools

import jax
import jax.numpy as jnp
from jax import lax
from jax.experimental import pallas as pl
from jax.experimental.pallas import tpu as pltpu
from jax.experimental.pallas import tpu_sc as plsc

_B, _D, _C, _E = 4096, 1000, 40, 64
_K = 8
_CP = 48
_NC, _NS = 2, 16
_NW = _NC * _NS
_QPW = _B // _NW
_BB = 256
_NEG = -1e30


def _score_topk_body(dfea_ref, dp_ref, v1_ref, u1_ref, dsim_ref, idx_ref):
    f32 = jnp.float32
    m_mat = lax.dot_general(u1_ref[...], v1_ref[...], (((0,), (0,)), ((), ())),
                            preferred_element_type=f32)
    g = jnp.dot(dfea_ref[...], m_mat, preferred_element_type=f32)
    p = lax.dot_general(g, dp_ref[...], (((1,), (1,)), ((), ())),
                        preferred_element_type=f32)
    p_max = jnp.max(p, axis=1, keepdims=True)
    p_exp = jnp.exp(p - p_max)
    p_sum = jnp.sum(p_exp, axis=1, keepdims=True)
    pd = p_exp / p_sum
    iota = lax.broadcasted_iota(jnp.int32, pd.shape, 1)
    vals, idxs = [], []
    for _ in range(_K):
        mv = jnp.max(pd, axis=1, keepdims=True)
        cand = jnp.where(pd == mv, iota, _D)
        am = jnp.min(cand, axis=1, keepdims=True)
        vals.append(mv)
        idxs.append(am)
        pd = jnp.where(iota == am, -1.0, pd)
    dsim_ref[...] = jnp.concatenate(vals, axis=1)
    idx_ref[...] = jnp.concatenate(idxs, axis=1)


_score_topk = pl.pallas_call(
    _score_topk_body,
    grid=(_B // _BB,),
    in_specs=[
        pl.BlockSpec((_BB, _E), lambda i: (i, 0)),
        pl.BlockSpec((_D, _E), lambda i: (0, 0)),
        pl.BlockSpec((32, _E), lambda i: (0, 0)),
        pl.BlockSpec((32, _E), lambda i: (0, 0)),
    ],
    out_specs=[
        pl.BlockSpec((_BB, _K), lambda i: (i, 0)),
        pl.BlockSpec((_BB, _K), lambda i: (i, 0)),
    ],
    out_shape=[
        jax.ShapeDtypeStruct((_B, _K), jnp.float32),
        jax.ShapeDtypeStruct((_B, _K), jnp.int32),
    ],
)


def _classify_body(cp_hbm, idx_hbm, ds_hbm, cf_hbm, out_hbm,
                   idxv, dsv, cfv, rowbuf0, rowbuf1, outv, sem0, sem1):
    f32 = jnp.float32
    wid = lax.axis_index("s") * _NC + lax.axis_index("c")
    base = wid * _QPW
    pltpu.sync_copy(idx_hbm.at[pl.ds(base, _QPW)], idxv)
    pltpu.sync_copy(ds_hbm.at[pl.ds(base, _QPW)], dsv)
    pltpu.sync_copy(cf_hbm.at[pl.ds(base, _QPW)], cfv)

    lane = lax.iota(jnp.int32, 16)
    cbase = [lane * _E, (lane + 16) * _E, jnp.minimum(lane + 32, _C - 1) * _E]
    padmask = lane >= (_C - 32)
    zero16 = jnp.zeros((16,), f32)
    nvecs = [jnp.full((16,), n, jnp.int32) for n in range(_K)]
    bufs = (rowbuf0, rowbuf1)
    sems = (sem0, sem1)

    pltpu.make_async_copy(cp_hbm.at[idxv.at[0]], rowbuf0, sem0).start()

    def _compute_q(q, rowbuf):
        def chunk_body(ch, carry):
            accs = list(carry[:3 * _K])
            col = list(carry[3 * _K:])
            cf_chunk = cfv[q, pl.ds(ch * 16, 16)]
            for k in range(16):
                ce = cf_chunk.at[jnp.full((16,), k, jnp.int32)].get(
                    mode="promise_in_bounds")
                for j in range(3):
                    for n in range(_K):
                        g = plsc.load_gather(rowbuf, [nvecs[n], col[j]])
                        accs[n * 3 + j] = accs[n * 3 + j] + g * ce
                col = [c + 1 for c in col]
            return tuple(accs) + tuple(col)

        init = tuple([zero16] * (3 * _K)) + tuple(cbase)
        res = lax.fori_loop(0, _E // 16, chunk_body, init)

        ds_row = dsv[q, pl.ds(0, 16)]
        oa = [zero16, zero16, zero16]
        for n in range(_K):
            a = [res[n * 3], res[n * 3 + 1],
                 jnp.where(padmask, _NEG, res[n * 3 + 2])]
            m = jnp.maximum(jnp.maximum(jnp.max(a[0]), jnp.max(a[1])),
                            jnp.max(a[2]))
            ex = [jnp.exp(v - m) for v in a]
            s = jnp.sum(ex[0]) + jnp.sum(ex[1]) + jnp.sum(ex[2])
            dsb = ds_row.at[nvecs[n]].get(mode="promise_in_bounds")
            w = dsb / jnp.full((16,), s, f32)
            oa = [o + w * e for o, e in zip(oa, ex)]
        oa[2] = jnp.where(padmask, _NEG, oa[2])
        m = jnp.maximum(jnp.maximum(jnp.max(oa[0]), jnp.max(oa[1])),
                        jnp.max(oa[2]))
        fx = [jnp.exp(v - m) for v in oa]
        s = jnp.sum(fx[0]) + jnp.sum(fx[1]) + jnp.sum(fx[2])
        outv[q, pl.ds(0, 16)] = fx[0] / s
        outv[q, pl.ds(16, 16)] = fx[1] / s
        outv[q, pl.ds(32, 16)] = fx[2] / s

    def q_body(i, carry):
        for p in range(2):
            q = 2 * i + p
            pltpu.make_async_copy(
                cp_hbm.at[idxv.at[q]], bufs[p], sems[p]).wait()

            @pl.when(q + 1 < _QPW)
            def _():
                pltpu.make_async_copy(
                    cp_hbm.at[idxv.at[q + 1]], bufs[1 - p], sems[1 - p]
                ).start()

            _compute_q(q, bufs[p])
        return carry

    lax.fori_loop(0, _QPW // 2, q_body, 0)
    pltpu.sync_copy(outv, out_hbm.at[pl.ds(base, _QPW)])


@functools.cache
def _make_classify():
    return pl.kernel(
        _classify_body,
        mesh=plsc.VectorSubcoreMesh(core_axis_name="c", subcore_axis_name="s"),
        out_type=jax.ShapeDtypeStruct((_B, _CP), jnp.float32),
        compiler_params=pltpu.CompilerParams(
            use_tc_tiling_on_sc=False, needs_layout_passes=False),
        scratch_types=[
            pltpu.VMEM((_QPW, _K), jnp.int32),
            pltpu.VMEM((_QPW, 16), jnp.float32),
            pltpu.VMEM((_QPW, _E), jnp.float32),
            pltpu.VMEM((_K, _C * _E), jnp.float32),
            pltpu.VMEM((_K, _C * _E), jnp.float32),
            pltpu.VMEM((_QPW, _CP), jnp.float32),
            pltpu.SemaphoreType.DMA,
            pltpu.SemaphoreType.DMA,
        ],
    )


def kernel(c_fea, d_fea, domain_p, class_p, V1, U1, num):
    del num
    dsim, idx = _score_topk(d_fea, domain_p, V1, U1)
    dsim16 = jnp.concatenate(
        [dsim, jnp.zeros((_B, 16 - _K), jnp.float32)], axis=1)
    cp2d = class_p.reshape(_D, _C * _E)
    out48 = _make_classify()(cp2d, idx, dsim16, c_fea)
    return out48[:, :_C]


# speedup vs baseline: 2.6173x; 2.3020x over previous
import functools

import jax
import jax.numpy as jnp
from jax import lax
from jax.experimental import pallas as pl
from jax.experimental.pallas import tpu as pltpu
from jax.experimental.pallas import tpu_sc as plsc

_B, _D, _C, _E = 4096, 1000, 40, 64
_K = 8
_CP = 48
_NC, _NS = 2, 16
_NW = _NC * _NS
_QPW = _B // _NW
_BB = 256
_NEG = -1e30


def _score_topk_body(dfea_ref, dp_ref, v1_ref, u1_ref, dsim_ref, idx_ref):
    f32 = jnp.float32
    m_mat = lax.dot_general(u1_ref[...], v1_ref[...], (((0,), (0,)), ((), ())),
                            preferred_element_type=f32)
    g = jnp.dot(dfea_ref[...], m_mat, preferred_element_type=f32)
    p = lax.dot_general(g, dp_ref[...], (((1,), (1,)), ((), ())),
                        preferred_element_type=f32)
    p_max = jnp.max(p, axis=1, keepdims=True)
    p_exp = jnp.exp(p - p_max)
    p_sum = jnp.sum(p_exp, axis=1, keepdims=True)
    pd = p_exp / p_sum
    iota = lax.broadcasted_iota(jnp.int32, pd.shape, 1)
    vals, idxs = [], []
    for _ in range(_K):
        mv = jnp.max(pd, axis=1, keepdims=True)
        cand = jnp.where(pd == mv, iota, _D)
        am = jnp.min(cand, axis=1, keepdims=True)
        vals.append(mv)
        idxs.append(am)
        pd = jnp.where(iota == am, -1.0, pd)
    dsim_ref[...] = jnp.concatenate(vals, axis=1)
    idx_ref[...] = jnp.concatenate(idxs, axis=1)


_score_topk = pl.pallas_call(
    _score_topk_body,
    grid=(_B // _BB,),
    in_specs=[
        pl.BlockSpec((_BB, _E), lambda i: (i, 0)),
        pl.BlockSpec((_D, _E), lambda i: (0, 0)),
        pl.BlockSpec((32, _E), lambda i: (0, 0)),
        pl.BlockSpec((32, _E), lambda i: (0, 0)),
    ],
    out_specs=[
        pl.BlockSpec((_BB, _K), lambda i: (i, 0)),
        pl.BlockSpec((_BB, _K), lambda i: (i, 0)),
    ],
    out_shape=[
        jax.ShapeDtypeStruct((_B, _K), jnp.float32),
        jax.ShapeDtypeStruct((_B, _K), jnp.int32),
    ],
)


def _transpose_body(cp_ref, out_ref):
    out_ref[...] = jnp.transpose(cp_ref[...], (0, 2, 1))


_transpose_cp = pl.pallas_call(
    _transpose_body,
    grid=(_D // 8,),
    in_specs=[pl.BlockSpec((8, _C, _E), lambda i: (i, 0, 0))],
    out_specs=pl.BlockSpec((8, _E, _C), lambda i: (i, 0, 0)),
    out_shape=jax.ShapeDtypeStruct((_D, _E, _C), jnp.float32),
)


def _classify_body(cp_hbm, idx_hbm, ds_hbm, cf_hbm, out_hbm,
                   idxv, dsv, cfv, rowbuf0, rowbuf1, outv, sem0, sem1):
    f32 = jnp.float32
    wid = lax.axis_index("s") * _NC + lax.axis_index("c")
    base = wid * _QPW
    pltpu.sync_copy(idx_hbm.at[pl.ds(base, _QPW)], idxv)
    pltpu.sync_copy(ds_hbm.at[pl.ds(base, _QPW)], dsv)
    pltpu.sync_copy(cf_hbm.at[pl.ds(base, _QPW)], cfv)

    lane = lax.iota(jnp.int32, 16)
    padmask = lane >= (_C - 32)
    zero16 = jnp.zeros((16,), f32)
    nvecs = [jnp.full((16,), n, jnp.int32) for n in range(_K)]
    bufs = (rowbuf0, rowbuf1)
    sems = (sem0, sem1)

    pltpu.make_async_copy(
        cp_hbm.at[idxv.at[0]], rowbuf0.at[pl.ds(0, _K)], sem0).start()

    def _compute_q(q, rowbuf):
        def chunk_body(ch, carry):
            accs = list(carry)
            cf_chunk = cfv[q, pl.ds(ch * 16, 16)]
            ebase = ch * (16 * _C)
            for k in range(16):
                ce = cf_chunk.at[jnp.full((16,), k, jnp.int32)].get(
                    mode="promise_in_bounds")
                off0 = ebase + k * _C
                for j in range(3):
                    for n in range(_K):
                        g = rowbuf[n, pl.ds(off0 + 16 * j, 16)]
                        accs[n * 3 + j] = accs[n * 3 + j] + g * ce
            return tuple(accs)

        init = tuple([zero16] * (3 * _K))
        res = lax.fori_loop(0, _E // 16, chunk_body, init)

        ds_row = dsv[q, pl.ds(0, 16)]
        oa = [zero16, zero16, zero16]
        for n in range(_K):
            a = [res[n * 3], res[n * 3 + 1],
                 jnp.where(padmask, _NEG, res[n * 3 + 2])]
            m = jnp.maximum(jnp.maximum(jnp.max(a[0]), jnp.max(a[1])),
                            jnp.max(a[2]))
            ex = [jnp.exp(v - m) for v in a]
            s = jnp.sum(ex[0]) + jnp.sum(ex[1]) + jnp.sum(ex[2])
            dsb = ds_row.at[nvecs[n]].get(mode="promise_in_bounds")
            w = dsb / jnp.full((16,), s, f32)
            oa = [o + w * e for o, e in zip(oa, ex)]
        oa[2] = jnp.where(padmask, _NEG, oa[2])
        m = jnp.maximum(jnp.maximum(jnp.max(oa[0]), jnp.max(oa[1])),
                        jnp.max(oa[2]))
        fx = [jnp.exp(v - m) for v in oa]
        s = jnp.sum(fx[0]) + jnp.sum(fx[1]) + jnp.sum(fx[2])
        outv[q, pl.ds(0, 16)] = fx[0] / s
        outv[q, pl.ds(16, 16)] = fx[1] / s
        outv[q, pl.ds(32, 16)] = fx[2] / s

    def q_body(i, carry):
        for p in range(2):
            q = 2 * i + p
            pltpu.make_async_copy(
                cp_hbm.at[idxv.at[q]], bufs[p].at[pl.ds(0, _K)],
                sems[p]).wait()

            @pl.when(q + 1 < _QPW)
            def _():
                pltpu.make_async_copy(
                    cp_hbm.at[idxv.at[q + 1]], bufs[1 - p].at[pl.ds(0, _K)],
                    sems[1 - p]
                ).start()

            _compute_q(q, bufs[p])
        return carry

    lax.fori_loop(0, _QPW // 2, q_body, 0)
    pltpu.sync_copy(outv, out_hbm.at[pl.ds(base, _QPW)])


@functools.cache
def _make_classify():
    return pl.kernel(
        _classify_body,
        mesh=plsc.VectorSubcoreMesh(core_axis_name="c", subcore_axis_name="s"),
        out_type=jax.ShapeDtypeStruct((_B, _CP), jnp.float32),
        compiler_params=pltpu.CompilerParams(
            use_tc_tiling_on_sc=False, needs_layout_passes=False),
        scratch_types=[
            pltpu.VMEM((_QPW, _K), jnp.int32),
            pltpu.VMEM((_QPW, 16), jnp.float32),
            pltpu.VMEM((_QPW, _E), jnp.float32),
            pltpu.VMEM((_K + 1, _C * _E), jnp.float32),
            pltpu.VMEM((_K + 1, _C * _E), jnp.float32),
            pltpu.VMEM((_QPW, _CP), jnp.float32),
            pltpu.SemaphoreType.DMA,
            pltpu.SemaphoreType.DMA,
        ],
    )


def kernel(c_fea, d_fea, domain_p, class_p, V1, U1, num):
    del num
    dsim, idx = _score_topk(d_fea, domain_p, V1, U1)
    dsim16 = jnp.concatenate(
        [dsim, jnp.zeros((_B, 16 - _K), jnp.float32)], axis=1)
    cpt2d = _transpose_cp(class_p).reshape(_D, _C * _E)
    out48 = _make_classify()(cpt2d, idx, dsim16, c_fea)
    return out48[:, :_C]


# speedup vs baseline: 3.2548x; 1.2436x over previous
import functools

import jax
import jax.numpy as jnp
from jax import lax
from jax.experimental import pallas as pl
from jax.experimental.pallas import tpu as pltpu
from jax.experimental.pallas import tpu_sc as plsc

_B, _D, _C, _E = 4096, 1000, 40, 64
_K = 8
_CP = 48
_NC, _NS = 2, 16
_NW = _NC * _NS
_QPW = _B // _NW
_BB = 256
_NEG = -1e30


def _score_topk_body(dfea_ref, dp_ref, v1_ref, u1_ref, dsim_ref, idx_ref):
    f32 = jnp.float32
    m_mat = lax.dot_general(u1_ref[...], v1_ref[...], (((0,), (0,)), ((), ())),
                            preferred_element_type=f32)
    g = jnp.dot(dfea_ref[...], m_mat, preferred_element_type=f32)
    p = lax.dot_general(g, dp_ref[...], (((1,), (1,)), ((), ())),
                        preferred_element_type=f32)
    p_max = jnp.max(p, axis=1, keepdims=True)
    p_exp = jnp.exp(p - p_max)
    p_sum = jnp.sum(p_exp, axis=1, keepdims=True)
    pd = p_exp / p_sum
    iota = lax.broadcasted_iota(jnp.int32, pd.shape, 1)
    vals, idxs = [], []
    for _ in range(_K):
        mv = jnp.max(pd, axis=1, keepdims=True)
        cand = jnp.where(pd == mv, iota, _D)
        am = jnp.min(cand, axis=1, keepdims=True)
        vals.append(mv)
        idxs.append(am)
        pd = jnp.where(iota == am, -1.0, pd)
    dsim_ref[...] = jnp.concatenate(vals, axis=1)
    idx_ref[...] = jnp.concatenate(idxs, axis=1)


_score_topk = pl.pallas_call(
    _score_topk_body,
    grid=(_B // _BB,),
    in_specs=[
        pl.BlockSpec((_BB, _E), lambda i: (i, 0)),
        pl.BlockSpec((_D, _E), lambda i: (0, 0)),
        pl.BlockSpec((32, _E), lambda i: (0, 0)),
        pl.BlockSpec((32, _E), lambda i: (0, 0)),
    ],
    out_specs=[
        pl.BlockSpec((_BB, _K), lambda i: (i, 0)),
        pl.BlockSpec((_BB, _K), lambda i: (i, 0)),
    ],
    out_shape=[
        jax.ShapeDtypeStruct((_B, _K), jnp.float32),
        jax.ShapeDtypeStruct((_B, _K), jnp.int32),
    ],
)


def _transpose_body(cp_ref, out_ref):
    out_ref[...] = jnp.transpose(cp_ref[...], (0, 2, 1))


_transpose_cp = pl.pallas_call(
    _transpose_body,
    grid=(_D // 8,),
    in_specs=[pl.BlockSpec((8, _C, _E), lambda i: (i, 0, 0))],
    out_specs=pl.BlockSpec((8, _E, _C), lambda i: (i, 0, 0)),
    out_shape=jax.ShapeDtypeStruct((_D, _E, _C), jnp.float32),
)


def _classify_body(cp_hbm, idx_hbm, ds_hbm, cf_hbm, out_hbm,
                   idxv, dsv, cfv, rowbuf0, rowbuf1, outv, sem0, sem1):
    f32 = jnp.float32
    wid = lax.axis_index("s") * _NC + lax.axis_index("c")
    base = wid * _QPW
    pltpu.sync_copy(idx_hbm.at[pl.ds(base, _QPW)], idxv)
    pltpu.sync_copy(ds_hbm.at[pl.ds(base, _QPW)], dsv)
    pltpu.sync_copy(cf_hbm.at[pl.ds(base, _QPW)], cfv)

    lane = lax.iota(jnp.int32, 16)
    padmask = lane < 8
    zero16 = jnp.zeros((16,), f32)
    nvecs = [jnp.full((16,), n, jnp.int32) for n in range(_K)]
    bufs = (rowbuf0, rowbuf1)
    sems = (sem0, sem1)

    pltpu.make_async_copy(
        cp_hbm.at[idxv.at[0]], rowbuf0.at[pl.ds(0, _K)], sem0).start()

    def _compute_q(q, rowbuf):
        ds_row = dsv[q, pl.ds(0, 16)]

        def n_body(n, oacc):
            acc = [zero16, zero16, zero16]
            for ch in range(_E // 16):
                cf_chunk = cfv[q, pl.ds(ch * 16, 16)]
                for k in range(16):
                    ce = cf_chunk.at[jnp.full((16,), k, jnp.int32)].get(
                        mode="promise_in_bounds")
                    off0 = (ch * 16 + k) * _C
                    for j, joff in enumerate((0, 16, 24)):
                        g = rowbuf[n, pl.ds(off0 + joff, 16)]
                        acc[j] = acc[j] + g * ce
            a2 = jnp.where(padmask, _NEG, acc[2])
            m = jnp.max(jnp.maximum(jnp.maximum(acc[0], acc[1]), a2))
            ex = [jnp.exp(acc[0] - m), jnp.exp(acc[1] - m), jnp.exp(a2 - m)]
            s = jnp.sum(ex[0] + ex[1] + ex[2])
            dsb = ds_row.at[jnp.full((16,), n, jnp.int32)].get(
                mode="promise_in_bounds")
            w = dsb / jnp.full((16,), s, f32)
            return (oacc[0] + w * ex[0], oacc[1] + w * ex[1],
                    oacc[2] + w * ex[2])

        oa = list(lax.fori_loop(0, _K, n_body, (zero16, zero16, zero16)))
        oa[2] = jnp.where(padmask, _NEG, oa[2])
        m = jnp.max(jnp.maximum(jnp.maximum(oa[0], oa[1]), oa[2]))
        fx = [jnp.exp(v - m) for v in oa]
        rs = jnp.ones((16,), f32) / jnp.full(
            (16,), jnp.sum(fx[0] + fx[1] + fx[2]), f32)
        f2 = fx[2].at[(lane & 7) + 8].get(mode="promise_in_bounds")
        outv[q, pl.ds(0, 16)] = fx[0] * rs
        outv[q, pl.ds(16, 16)] = fx[1] * rs
        outv[q, pl.ds(32, 16)] = f2 * rs

    def q_body(i, carry):
        for p in range(2):
            q = 2 * i + p
            pltpu.make_async_copy(
                cp_hbm.at[idxv.at[q]], bufs[p].at[pl.ds(0, _K)],
                sems[p]).wait()

            @pl.when(q + 1 < _QPW)
            def _():
                pltpu.make_async_copy(
                    cp_hbm.at[idxv.at[q + 1]], bufs[1 - p].at[pl.ds(0, _K)],
                    sems[1 - p]
                ).start()

            _compute_q(q, bufs[p])
        return carry

    lax.fori_loop(0, _QPW // 2, q_body, 0)
    pltpu.sync_copy(outv, out_hbm.at[pl.ds(base, _QPW)])


@functools.cache
def _make_classify():
    return pl.kernel(
        _classify_body,
        mesh=plsc.VectorSubcoreMesh(core_axis_name="c", subcore_axis_name="s"),
        out_type=jax.ShapeDtypeStruct((_B, _CP), jnp.float32),
        compiler_params=pltpu.CompilerParams(
            use_tc_tiling_on_sc=False, needs_layout_passes=False),
        scratch_types=[
            pltpu.VMEM((_QPW, _K), jnp.int32),
            pltpu.VMEM((_QPW, 16), jnp.float32),
            pltpu.VMEM((_QPW, _E), jnp.float32),
            pltpu.VMEM((_K + 1, _C * _E), jnp.float32),
            pltpu.VMEM((_K + 1, _C * _E), jnp.float32),
            pltpu.VMEM((_QPW, _CP), jnp.float32),
            pltpu.SemaphoreType.DMA,
            pltpu.SemaphoreType.DMA,
        ],
    )


def kernel(c_fea, d_fea, domain_p, class_p, V1, U1, num):
    del num
    dsim, idx = _score_topk(d_fea, domain_p, V1, U1)
    dsim16 = jnp.concatenate(
        [dsim, jnp.zeros((_B, 16 - _K), jnp.float32)], axis=1)
    cpt2d = _transpose_cp(class_p).reshape(_D, _C * _E)
    out48 = _make_classify()(cpt2d, idx, dsim16, c_fea)
    return out48[:, :_C]
